# baseline (speedup 1.0000x reference)
import functools

import jax
import jax.numpy as jnp
from jax import lax
from jax.experimental import pallas as pl
from jax.experimental.pallas import tpu as pltpu
from jax.experimental.pallas import tpu_sc as plsc

N = 50000
T = 20
E = 800000
H = 64
G = 4 * H
OUT = 16
NP = 51200
EP = 802816
BN = 256
WIN = 128
NSUB = 16
NCORE = 2
HH = H // 2
SLICE = NP // NSUB

_f32 = jnp.float32


def _sc_mesh():
  return plsc.VectorSubcoreMesh(core_axis_name="core",
                                subcore_axis_name="subcore")


_SC_PARAMS = pltpu.CompilerParams(use_tc_tiling_on_sc=False,
                                  needs_layout_passes=False)


def _emb_gather(emb, xt):
  @functools.partial(
      pl.kernel,
      out_type=jax.ShapeDtypeStruct((T, NP, H), jnp.bfloat16),
      mesh=_sc_mesh(),
      compiler_params=_SC_PARAMS,
  )
  def k(emb_hbm, x_hbm, o_hbm):
    def body(i_vmem, o_vmem):
      pltpu.sync_copy(emb_hbm.at[i_vmem.at[0]], o_vmem.at[0])

    pltpu.emit_pipeline(
        body,
        grid=(T, NP // WIN),
        in_specs=[pl.BlockSpec((1, WIN), lambda t, j: (t, j))],
        out_specs=[pl.BlockSpec((1, WIN, H), lambda t, j: (t, j, 0))],
        core_axis_name=("core", "subcore"),
        dimension_semantics=(pltpu.PARALLEL, pltpu.PARALLEL),
    )(x_hbm, o_hbm)

  return k(emb, xt)


def _deg_pass(dst, w):
  @functools.partial(
      pl.kernel,
      out_type=jax.ShapeDtypeStruct((NCORE, NP), _f32),
      mesh=_sc_mesh(),
      compiler_params=_SC_PARAMS,
      scratch_types=[
          pltpu.VMEM_SHARED((NP,), _f32),
          pltpu.VMEM((SLICE,), _f32),
      ],
  )
  def k(dst_hbm, w_hbm, o_hbm, acc, zbuf):
    core = lax.axis_index("core")
    sub = lax.axis_index("subcore")

    @pl.loop(0, SLICE // 16)
    def _(i):
      zbuf[pl.ds(i * 16, 16)] = jnp.zeros((16,), _f32)

    pltpu.sync_copy(zbuf, acc.at[pl.ds(sub * SLICE, SLICE)])
    plsc.subcore_barrier()

    def body(d_vmem, w_vmem):
      pltpu.sync_copy(w_vmem.at[0], acc.at[d_vmem.at[0]], add=True)

    pltpu.emit_pipeline(
        body,
        grid=(EP // WIN,),
        in_specs=[
            pl.BlockSpec((1, WIN), lambda i: (0, i)),
            pl.BlockSpec((1, WIN), lambda i: (0, i)),
        ],
        core_axis_name=("core", "subcore"),
        dimension_semantics=(pltpu.PARALLEL,),
    )(dst_hbm, w_hbm)

    plsc.subcore_barrier()
    pltpu.sync_copy(acc.at[pl.ds(sub * SLICE, SLICE)],
                    o_hbm.at[core, pl.ds(sub * SLICE, SLICE)])

  return k(dst, w)


SB = 4
NBLK = EP // WIN // NSUB // SB
NPAIR = NBLK // 2


def _msg_pass(hwp0, hwp1, src2, dst2, w2):
  @functools.partial(
      pl.kernel,
      out_type=jax.ShapeDtypeStruct((NCORE, NP, HH), _f32),
      mesh=_sc_mesh(),
      compiler_params=_SC_PARAMS,
      scratch_types=[
          pltpu.VMEM_SHARED((NP, HH), _f32),
          pltpu.VMEM((SB, WIN, HH), _f32),
          pltpu.VMEM((SB, WIN), jnp.int32),
          pltpu.VMEM((SB, WIN), jnp.int32),
          pltpu.VMEM((SB, WIN), _f32),
          pltpu.VMEM((SB, WIN), jnp.int32),
          pltpu.VMEM((SB, WIN), jnp.int32),
          pltpu.VMEM((SB, WIN), _f32),
          pltpu.SemaphoreType.DMA,
          pltpu.SemaphoreType.DMA,
          pltpu.SemaphoreType.DMA,
      ],
  )
  def k(t0_hbm, t1_hbm, s_hbm, d_hbm, w_hbm, o_hbm,
        acc, rbuf, s0, d0, w0, s1, d1, w1, sem_i, sem_g, sem_s):
    core = lax.axis_index("core")
    sub = lax.axis_index("subcore")
    del s0, d0, w0, s1, d1, w1, sem_i

    @pl.loop(0, WIN)
    def _(i):
      rbuf[0, i, pl.ds(0, 16)] = jnp.zeros((16,), _f32)
      rbuf[0, i, pl.ds(16, 16)] = jnp.zeros((16,), _f32)

    @pl.loop(0, SLICE // WIN)
    def _(j):
      pltpu.sync_copy(rbuf.at[0], acc.at[pl.ds(sub * SLICE + j * WIN, WIN)])

    plsc.subcore_barrier()

    def body(s_vmem, d_vmem, w_vmem):
      for j in range(SB):
        @pl.when(core == 0)
        def _():
          pltpu.sync_copy(t0_hbm.at[s_vmem.at[j]], rbuf.at[j])

        @pl.when(core == 1)
        def _():
          pltpu.sync_copy(t1_hbm.at[s_vmem.at[j]], rbuf.at[j])

      for j in range(SB):
        @pl.loop(0, WIN // 16)
        def _(kk):
          base = kk * 16
          for i in range(16):
            we = plsc.load_gather(
                w_vmem, [jnp.full((16,), j, jnp.int32),
                         jnp.full((16,), base + i, jnp.int32)])
            rbuf[j, base + i, pl.ds(0, 16)] = (
                rbuf[j, base + i, pl.ds(0, 16)] * we)
            rbuf[j, base + i, pl.ds(16, 16)] = (
                rbuf[j, base + i, pl.ds(16, 16)] * we)

        pltpu.sync_copy(rbuf.at[j], acc.at[d_vmem.at[j]], add=True)

    pltpu.emit_pipeline(
        body,
        grid=(EP // WIN // SB,),
        in_specs=[
            pl.BlockSpec((SB, WIN), lambda i: (i, 0)),
            pl.BlockSpec((SB, WIN), lambda i: (i, 0)),
            pl.BlockSpec((SB, WIN), lambda i: (i, 0)),
        ],
        core_axis_name="subcore",
        dimension_semantics=(pltpu.PARALLEL,),
    )(s_hbm, d_hbm, w_hbm)

    plsc.subcore_barrier()
    pltpu.sync_copy(acc.at[pl.ds(sub * SLICE, SLICE)],
                    o_hbm.at[core, pl.ds(sub * SLICE, SLICE)])

  return k(hwp0, hwp1, src2, dst2, w2)


def _lstm_tc(xe, lengths2, deg2, wcat, bias, w1):
  def body(xe_ref, len_ref, deg_ref, wcat_ref, b_ref, w1_ref,
           dis_ref, hwp0_ref, hwp1_ref):
    lens = len_ref[...]
    wc = wcat_ref[...]
    b = b_ref[...]
    h = jnp.zeros((BN, H), _f32)
    c = jnp.zeros((BN, H), _f32)
    hlast = jnp.zeros((BN, H), _f32)
    for t in range(T):
      cat = jnp.concatenate([xe_ref[t].astype(_f32), h], axis=1)
      g = jnp.dot(cat, wc, preferred_element_type=_f32) + b
      i = jax.nn.sigmoid(g[:, 0:H])
      f = jax.nn.sigmoid(g[:, H:2 * H])
      gg = jnp.tanh(g[:, 2 * H:3 * H])
      o = jax.nn.sigmoid(g[:, 3 * H:4 * H])
      c = f * c + i * gg
      h = o * jnp.tanh(c)
      hlast = jnp.where(lens == t + 1, h, hlast)
    deg = deg_ref[:, 0:1] + deg_ref[:, 1:2] + 1.0
    dis = jnp.where(deg > 0, lax.rsqrt(deg), 0.0)
    dis_ref[...] = dis
    hwp = dis * jnp.dot(hlast, w1_ref[...], preferred_element_type=_f32)
    hwp0_ref[...] = hwp[:, 0:HH]
    hwp1_ref[...] = hwp[:, HH:H]

  return pl.pallas_call(
      body,
      grid=(NP // BN,),
      in_specs=[
          pl.BlockSpec((T, BN, H), lambda b: (0, b, 0)),
          pl.BlockSpec((BN, 1), lambda b: (b, 0)),
          pl.BlockSpec((BN, 2), lambda b: (b, 0)),
          pl.BlockSpec((2 * H, G), lambda b: (0, 0)),
          pl.BlockSpec((1, G), lambda b: (0, 0)),
          pl.BlockSpec((H, H), lambda b: (0, 0)),
      ],
      out_specs=[
          pl.BlockSpec((BN, 1), lambda b: (b, 0)),
          pl.BlockSpec((BN, HH), lambda b: (b, 0)),
          pl.BlockSpec((BN, HH), lambda b: (b, 0)),
      ],
      out_shape=[
          jax.ShapeDtypeStruct((NP, 1), _f32),
          jax.ShapeDtypeStruct((NP, HH), _f32),
          jax.ShapeDtypeStruct((NP, HH), _f32),
      ],
  )(xe, lengths2, deg2, wcat, bias, w1)


def _combine_mid_tc(msg, hwp0, hwp1, dis, wnext, bias):
  def body(m0_ref, m1_ref, p0_ref, p1_ref, dis_ref, w_ref, b_ref,
           q0_ref, q1_ref):
    d = dis_ref[...]
    pre = jnp.concatenate(
        [m0_ref[0] + p0_ref[...], m1_ref[0] + p1_ref[...]], axis=1)
    hcur = jnp.maximum(d * pre + b_ref[...], 0.0)
    hwp = d * jnp.dot(hcur, w_ref[...], preferred_element_type=_f32)
    q0_ref[...] = hwp[:, 0:HH]
    q1_ref[...] = hwp[:, HH:H]

  return pl.pallas_call(
      body,
      grid=(NP // BN,),
      in_specs=[
          pl.BlockSpec((1, BN, HH), lambda b: (0, b, 0)),
          pl.BlockSpec((1, BN, HH), lambda b: (1, b, 0)),
          pl.BlockSpec((BN, HH), lambda b: (b, 0)),
          pl.BlockSpec((BN, HH), lambda b: (b, 0)),
          pl.BlockSpec((BN, 1), lambda b: (b, 0)),
          pl.BlockSpec((H, H), lambda b: (0, 0)),
          pl.BlockSpec((1, H), lambda b: (0, 0)),
      ],
      out_specs=[
          pl.BlockSpec((BN, HH), lambda b: (b, 0)),
          pl.BlockSpec((BN, HH), lambda b: (b, 0)),
      ],
      out_shape=[
          jax.ShapeDtypeStruct((NP, HH), _f32),
          jax.ShapeDtypeStruct((NP, HH), _f32),
      ],
  )(msg, msg, hwp0, hwp1, dis, wnext, bias)


def _combine_final_tc(msg, hwp0, hwp1, dis, wfc, bias, bfc):
  def body(m0_ref, m1_ref, p0_ref, p1_ref, dis_ref, w_ref, b_ref,
           bf_ref, o_ref):
    d = dis_ref[...]
    pre = jnp.concatenate(
        [m0_ref[0] + p0_ref[...], m1_ref[0] + p1_ref[...]], axis=1)
    hcur = jnp.maximum(d * pre + b_ref[...], 0.0)
    o_ref[...] = jnp.dot(
        hcur, w_ref[...], preferred_element_type=_f32) + bf_ref[...]

  return pl.pallas_call(
      body,
      grid=(NP // BN,),
      in_specs=[
          pl.BlockSpec((1, BN, HH), lambda b: (0, b, 0)),
          pl.BlockSpec((1, BN, HH), lambda b: (1, b, 0)),
          pl.BlockSpec((BN, HH), lambda b: (b, 0)),
          pl.BlockSpec((BN, HH), lambda b: (b, 0)),
          pl.BlockSpec((BN, 1), lambda b: (b, 0)),
          pl.BlockSpec((H, OUT), lambda b: (0, 0)),
          pl.BlockSpec((1, H), lambda b: (0, 0)),
          pl.BlockSpec((1, OUT), lambda b: (0, 0)),
      ],
      out_specs=[pl.BlockSpec((BN, OUT), lambda b: (b, 0))],
      out_shape=[jax.ShapeDtypeStruct((NP, OUT), _f32)],
  )(msg, msg, hwp0, hwp1, dis, wfc, bias, bfc)


def kernel(x, edge_index, edge_weight, lengths, emb, W_ih, W_hh, b_ih, b_hh,
           W1, b1, W2, b2, Wfc, bfc):
  xt = jnp.pad(x.T.astype(jnp.int32), ((0, 0), (0, NP - N)))
  lengths2 = jnp.pad(lengths.astype(jnp.int32), (0, NP - N),
                     constant_values=1).reshape(NP, 1)
  src = jnp.pad(edge_index[0].astype(jnp.int32), (0, EP - E)).reshape(1, EP)
  dst = jnp.pad(edge_index[1].astype(jnp.int32), (0, EP - E)).reshape(1, EP)
  w = jnp.pad(edge_weight, (0, EP - E)).reshape(1, EP)
  src2 = src.reshape(EP // WIN, WIN)
  dst2 = dst.reshape(EP // WIN, WIN)
  w2 = w.reshape(EP // WIN, WIN)
  wcat = jnp.concatenate([W_ih.T, W_hh.T], axis=0)
  bias = (b_ih + b_hh).reshape(1, G)

  xe = _emb_gather(emb.astype(jnp.bfloat16), xt)
  degp = _deg_pass(dst, w)
  deg2 = degp.T

  dis, hwp0, hwp1 = _lstm_tc(xe, lengths2, deg2, wcat, bias, W1)

  msg1 = _msg_pass(hwp0, hwp1, src2, dst2, w2)
  hwp2_0, hwp2_1 = _combine_mid_tc(msg1, hwp0, hwp1, dis, W2,
                                   b1.reshape(1, H))

  msg2 = _msg_pass(hwp2_0, hwp2_1, src2, dst2, w2)
  out = _combine_final_tc(msg2, hwp2_0, hwp2_1, dis, Wfc,
                          b2.reshape(1, H), bfc.reshape(1, OUT))[0]

  return out[:N]


# speedup vs baseline: 1.9988x; 1.9988x over previous
import functools

import jax
import jax.numpy as jnp
from jax import lax
from jax.experimental import pallas as pl
from jax.experimental.pallas import tpu as pltpu
from jax.experimental.pallas import tpu_sc as plsc

N = 50000
T = 20
E = 800000
H = 64
G = 4 * H
OUT = 16
NP = 51200
EP = 802816
BN = 256
WIN = 128
NSUB = 16
NCORE = 2
HH = H // 2
SLICE = NP // NSUB

_f32 = jnp.float32


def _sc_mesh():
  return plsc.VectorSubcoreMesh(core_axis_name="core",
                                subcore_axis_name="subcore")


_SC_PARAMS = pltpu.CompilerParams(use_tc_tiling_on_sc=False,
                                  needs_layout_passes=False)


def _emb_gather(emb, xt):
  @functools.partial(
      pl.kernel,
      out_type=jax.ShapeDtypeStruct((T, NP, H), jnp.bfloat16),
      mesh=_sc_mesh(),
      compiler_params=_SC_PARAMS,
  )
  def k(emb_hbm, x_hbm, o_hbm):
    def body(i_vmem, o_vmem):
      pltpu.sync_copy(emb_hbm.at[i_vmem.at[0]], o_vmem.at[0])

    pltpu.emit_pipeline(
        body,
        grid=(T, NP // WIN),
        in_specs=[pl.BlockSpec((1, WIN), lambda t, j: (t, j))],
        out_specs=[pl.BlockSpec((1, WIN, H), lambda t, j: (t, j, 0))],
        core_axis_name=("core", "subcore"),
        dimension_semantics=(pltpu.PARALLEL, pltpu.PARALLEL),
    )(x_hbm, o_hbm)

  return k(emb, xt)


def _deg_pass(dst, w):
  @functools.partial(
      pl.kernel,
      out_type=jax.ShapeDtypeStruct((NCORE, NP), _f32),
      mesh=_sc_mesh(),
      compiler_params=_SC_PARAMS,
      scratch_types=[
          pltpu.VMEM_SHARED((NP,), _f32),
          pltpu.VMEM((SLICE,), _f32),
      ],
  )
  def k(dst_hbm, w_hbm, o_hbm, acc, zbuf):
    core = lax.axis_index("core")
    sub = lax.axis_index("subcore")

    @pl.loop(0, SLICE // 16)
    def _(i):
      zbuf[pl.ds(i * 16, 16)] = jnp.zeros((16,), _f32)

    pltpu.sync_copy(zbuf, acc.at[pl.ds(sub * SLICE, SLICE)])
    plsc.subcore_barrier()

    def body(d_vmem, w_vmem):
      pltpu.sync_copy(w_vmem.at[0], acc.at[d_vmem.at[0]], add=True)

    pltpu.emit_pipeline(
        body,
        grid=(EP // WIN,),
        in_specs=[
            pl.BlockSpec((1, WIN), lambda i: (0, i)),
            pl.BlockSpec((1, WIN), lambda i: (0, i)),
        ],
        core_axis_name=("core", "subcore"),
        dimension_semantics=(pltpu.PARALLEL,),
    )(dst_hbm, w_hbm)

    plsc.subcore_barrier()
    pltpu.sync_copy(acc.at[pl.ds(sub * SLICE, SLICE)],
                    o_hbm.at[core, pl.ds(sub * SLICE, SLICE)])

  return k(dst, w)


def _msg_pass(hwp0, hwp1, src, dst, w):
  ZR = WIN

  @functools.partial(
      pl.kernel,
      out_type=jax.ShapeDtypeStruct((NCORE, NP, HH), _f32),
      mesh=_sc_mesh(),
      compiler_params=_SC_PARAMS,
      scratch_types=[
          pltpu.VMEM_SHARED((NP, HH), _f32),
          pltpu.VMEM((WIN, HH), _f32),
          pltpu.VMEM((ZR, HH), _f32),
      ],
  )
  def k(t0_hbm, t1_hbm, s_hbm, d_hbm, w_hbm, o_hbm, acc, rows, zbuf):
    core = lax.axis_index("core")
    sub = lax.axis_index("subcore")

    @pl.loop(0, ZR)
    def _(i):
      zbuf[i, pl.ds(0, 16)] = jnp.zeros((16,), _f32)
      zbuf[i, pl.ds(16, 16)] = jnp.zeros((16,), _f32)

    @pl.loop(0, SLICE // ZR)
    def _(j):
      pltpu.sync_copy(zbuf, acc.at[pl.ds(sub * SLICE + j * ZR, ZR)])

    plsc.subcore_barrier()

    def body(s_vmem, d_vmem, w_vmem):
      @pl.when(core == 0)
      def _():
        pltpu.sync_copy(t0_hbm.at[s_vmem.at[0]], rows)

      @pl.when(core == 1)
      def _():
        pltpu.sync_copy(t1_hbm.at[s_vmem.at[0]], rows)

      zero16 = jnp.zeros((16,), jnp.int32)

      @pl.loop(0, WIN // 16)
      def _(k):
        base = k * 16
        for j in range(16):
          we = plsc.load_gather(
              w_vmem, [zero16, jnp.full((16,), base + j, jnp.int32)])
          rows[base + j, pl.ds(0, 16)] = rows[base + j, pl.ds(0, 16)] * we
          rows[base + j, pl.ds(16, 16)] = rows[base + j, pl.ds(16, 16)] * we

      pltpu.sync_copy(rows, acc.at[d_vmem.at[0]], add=True)

    pltpu.emit_pipeline(
        body,
        grid=(EP // WIN,),
        in_specs=[
            pl.BlockSpec((1, WIN), lambda i: (0, i)),
            pl.BlockSpec((1, WIN), lambda i: (0, i)),
            pl.BlockSpec((1, WIN), lambda i: (0, i)),
        ],
        core_axis_name="subcore",
        dimension_semantics=(pltpu.PARALLEL,),
    )(s_hbm, d_hbm, w_hbm)

    plsc.subcore_barrier()
    pltpu.sync_copy(acc.at[pl.ds(sub * SLICE, SLICE)],
                    o_hbm.at[core, pl.ds(sub * SLICE, SLICE)])

  return k(hwp0, hwp1, src, dst, w)


def _lstm_tc(xe, lengths2, deg2, wcat, bias, w1):
  def body(xe_ref, len_ref, deg_ref, wcat_ref, b_ref, w1_ref,
           dis_ref, hwp0_ref, hwp1_ref):
    lens = len_ref[...]
    wc = wcat_ref[...]
    b = b_ref[...]
    h = jnp.zeros((BN, H), _f32)
    c = jnp.zeros((BN, H), _f32)
    hlast = jnp.zeros((BN, H), _f32)
    for t in range(T):
      cat = jnp.concatenate([xe_ref[t].astype(_f32), h], axis=1)
      g = jnp.dot(cat, wc, preferred_element_type=_f32) + b
      i = jax.nn.sigmoid(g[:, 0:H])
      f = jax.nn.sigmoid(g[:, H:2 * H])
      gg = jnp.tanh(g[:, 2 * H:3 * H])
      o = jax.nn.sigmoid(g[:, 3 * H:4 * H])
      c = f * c + i * gg
      h = o * jnp.tanh(c)
      hlast = jnp.where(lens == t + 1, h, hlast)
    deg = deg_ref[:, 0:1] + deg_ref[:, 1:2] + 1.0
    dis = jnp.where(deg > 0, lax.rsqrt(deg), 0.0)
    dis_ref[...] = dis
    hwp = dis * jnp.dot(hlast, w1_ref[...], preferred_element_type=_f32)
    hwp0_ref[...] = hwp[:, 0:HH]
    hwp1_ref[...] = hwp[:, HH:H]

  return pl.pallas_call(
      body,
      grid=(NP // BN,),
      in_specs=[
          pl.BlockSpec((T, BN, H), lambda b: (0, b, 0)),
          pl.BlockSpec((BN, 1), lambda b: (b, 0)),
          pl.BlockSpec((BN, 2), lambda b: (b, 0)),
          pl.BlockSpec((2 * H, G), lambda b: (0, 0)),
          pl.BlockSpec((1, G), lambda b: (0, 0)),
          pl.BlockSpec((H, H), lambda b: (0, 0)),
      ],
      out_specs=[
          pl.BlockSpec((BN, 1), lambda b: (b, 0)),
          pl.BlockSpec((BN, HH), lambda b: (b, 0)),
          pl.BlockSpec((BN, HH), lambda b: (b, 0)),
      ],
      out_shape=[
          jax.ShapeDtypeStruct((NP, 1), _f32),
          jax.ShapeDtypeStruct((NP, HH), _f32),
          jax.ShapeDtypeStruct((NP, HH), _f32),
      ],
  )(xe, lengths2, deg2, wcat, bias, w1)


def _combine_mid_tc(msg, hwp0, hwp1, dis, wnext, bias):
  def body(m0_ref, m1_ref, p0_ref, p1_ref, dis_ref, w_ref, b_ref,
           q0_ref, q1_ref):
    d = dis_ref[...]
    pre = jnp.concatenate(
        [m0_ref[0] + p0_ref[...], m1_ref[0] + p1_ref[...]], axis=1)
    hcur = jnp.maximum(d * pre + b_ref[...], 0.0)
    hwp = d * jnp.dot(hcur, w_ref[...], preferred_element_type=_f32)
    q0_ref[...] = hwp[:, 0:HH]
    q1_ref[...] = hwp[:, HH:H]

  return pl.pallas_call(
      body,
      grid=(NP // BN,),
      in_specs=[
          pl.BlockSpec((1, BN, HH), lambda b: (0, b, 0)),
          pl.BlockSpec((1, BN, HH), lambda b: (1, b, 0)),
          pl.BlockSpec((BN, HH), lambda b: (b, 0)),
          pl.BlockSpec((BN, HH), lambda b: (b, 0)),
          pl.BlockSpec((BN, 1), lambda b: (b, 0)),
          pl.BlockSpec((H, H), lambda b: (0, 0)),
          pl.BlockSpec((1, H), lambda b: (0, 0)),
      ],
      out_specs=[
          pl.BlockSpec((BN, HH), lambda b: (b, 0)),
          pl.BlockSpec((BN, HH), lambda b: (b, 0)),
      ],
      out_shape=[
          jax.ShapeDtypeStruct((NP, HH), _f32),
          jax.ShapeDtypeStruct((NP, HH), _f32),
      ],
  )(msg, msg, hwp0, hwp1, dis, wnext, bias)


def _combine_final_tc(msg, hwp0, hwp1, dis, wfc, bias, bfc):
  def body(m0_ref, m1_ref, p0_ref, p1_ref, dis_ref, w_ref, b_ref,
           bf_ref, o_ref):
    d = dis_ref[...]
    pre = jnp.concatenate(
        [m0_ref[0] + p0_ref[...], m1_ref[0] + p1_ref[...]], axis=1)
    hcur = jnp.maximum(d * pre + b_ref[...], 0.0)
    o_ref[...] = jnp.dot(
        hcur, w_ref[...], preferred_element_type=_f32) + bf_ref[...]

  return pl.pallas_call(
      body,
      grid=(NP // BN,),
      in_specs=[
          pl.BlockSpec((1, BN, HH), lambda b: (0, b, 0)),
          pl.BlockSpec((1, BN, HH), lambda b: (1, b, 0)),
          pl.BlockSpec((BN, HH), lambda b: (b, 0)),
          pl.BlockSpec((BN, HH), lambda b: (b, 0)),
          pl.BlockSpec((BN, 1), lambda b: (b, 0)),
          pl.BlockSpec((H, OUT), lambda b: (0, 0)),
          pl.BlockSpec((1, H), lambda b: (0, 0)),
          pl.BlockSpec((1, OUT), lambda b: (0, 0)),
      ],
      out_specs=[pl.BlockSpec((BN, OUT), lambda b: (b, 0))],
      out_shape=[jax.ShapeDtypeStruct((NP, OUT), _f32)],
  )(msg, msg, hwp0, hwp1, dis, wfc, bias, bfc)


def kernel(x, edge_index, edge_weight, lengths, emb, W_ih, W_hh, b_ih, b_hh,
           W1, b1, W2, b2, Wfc, bfc):
  xt = jnp.pad(x.T.astype(jnp.int32), ((0, 0), (0, NP - N)))
  lengths2 = jnp.pad(lengths.astype(jnp.int32), (0, NP - N),
                     constant_values=1).reshape(NP, 1)
  src = jnp.pad(edge_index[0].astype(jnp.int32), (0, EP - E)).reshape(1, EP)
  dst = jnp.pad(edge_index[1].astype(jnp.int32), (0, EP - E)).reshape(1, EP)
  w = jnp.pad(edge_weight, (0, EP - E)).reshape(1, EP)
  wcat = jnp.concatenate([W_ih.T, W_hh.T], axis=0)
  bias = (b_ih + b_hh).reshape(1, G)

  xe = _emb_gather(emb.astype(jnp.bfloat16), xt)
  degp = _deg_pass(dst, w)
  deg2 = degp.T

  dis, hwp0, hwp1 = _lstm_tc(xe, lengths2, deg2, wcat, bias, W1)

  msg1 = _msg_pass(hwp0, hwp1, src, dst, w)
  hwp2_0, hwp2_1 = _combine_mid_tc(msg1, hwp0, hwp1, dis, W2,
                                   b1.reshape(1, H))

  msg2 = _msg_pass(hwp2_0, hwp2_1, src, dst, w)
  out = _combine_final_tc(msg2, hwp2_0, hwp2_1, dis, Wfc,
                          b2.reshape(1, H), bfc.reshape(1, OUT))[0]

  return out[:N]
